# Initial kernel scaffold; baseline (speedup 1.0000x reference)
import jax
import jax.numpy as jnp
from jax.experimental import pallas as pl


def kernel(sequence, attrs_idxs, token_table, pos_table, attr_table0, attr_table1):
    raise NotImplementedError("write your pallas kernel here")


# speedup vs baseline: 1.5463x; 1.5463x over previous
import functools

import jax
import jax.numpy as jnp
from jax import lax
from jax.experimental import pallas as pl
from jax.experimental.pallas import tpu as pltpu
from jax.experimental.pallas import tpu_sc as plsc

_B, _L, _V, _E, _A = 1024, 200, 100000, 128, 8
_NC, _NS = 2, 16
_NW = _NC * _NS
_N = _B * _L
_TPW = _N // _NW
_STEP = 128
_NSTEP = _TPW // _STEP


def _body(seq_hbm, a0_hbm, a1_hbm, pidx_hbm,
          tok_hbm, pos_hbm, at0_hbm, at1_hbm,
          out_hbm,
          seq_v, a0_v, a1_v, pidx_v, rows_v, sem):
  c = lax.axis_index("c")
  s = lax.axis_index("s")
  wid = s * _NC + c

  pltpu.sync_copy(seq_hbm.at[wid], seq_v)
  pltpu.sync_copy(a0_hbm.at[wid], a0_v)
  pltpu.sync_copy(a1_hbm.at[wid], a1_v)
  pltpu.sync_copy(pidx_hbm.at[wid], pidx_v)

  def step(j, carry):
    pltpu.async_copy(tok_hbm.at[seq_v.at[j]], rows_v, sem).wait()
    d0 = pltpu.async_copy(at0_hbm.at[a0_v.at[j]], rows_v, sem, add=True)
    d1 = pltpu.async_copy(at1_hbm.at[a1_v.at[j]], rows_v, sem, add=True)
    dp = pltpu.async_copy(pos_hbm.at[pidx_v.at[j]], rows_v, sem, add=True)
    d0.wait()
    d1.wait()
    dp.wait()
    base = wid * _TPW + j * _STEP
    pltpu.sync_copy(rows_v, out_hbm.at[pl.ds(base, _STEP)])
    return carry

  lax.fori_loop(0, _NSTEP, step, 0)


@jax.jit
def _embed(seq, a0, a1, pidx, token_table, pos_table, attr_table0, attr_table1):
  mesh = plsc.VectorSubcoreMesh(core_axis_name="c", subcore_axis_name="s")
  return pl.kernel(
      _body,
      out_type=jax.ShapeDtypeStruct((_N, _E), jnp.float32),
      mesh=mesh,
      scratch_types=[
          pltpu.VMEM((_NSTEP, _STEP), jnp.int32),
          pltpu.VMEM((_NSTEP, _STEP), jnp.int32),
          pltpu.VMEM((_NSTEP, _STEP), jnp.int32),
          pltpu.VMEM((_NSTEP, _STEP), jnp.int32),
          pltpu.VMEM((_STEP, _E), jnp.float32),
          pltpu.SemaphoreType.DMA,
      ],
  )(seq, a0, a1, pidx, token_table, pos_table, attr_table0, attr_table1)


def kernel(sequence, attrs_idxs, token_table, pos_table, attr_table0, attr_table1):
  seq = sequence.astype(jnp.int32).reshape(_NW, _NSTEP, _STEP)
  a0 = attrs_idxs[0].astype(jnp.int32).reshape(_NW, _NSTEP, _STEP)
  a1 = attrs_idxs[1].astype(jnp.int32).reshape(_NW, _NSTEP, _STEP)
  pidx = jnp.broadcast_to(
      jnp.arange(_L, dtype=jnp.int32), (_B, _L)).reshape(_NW, _NSTEP, _STEP)
  out = _embed(seq, a0, a1, pidx,
               token_table, pos_table, attr_table0, attr_table1)
  return out.reshape(_B, _L, _E)
